# baseline (speedup 1.0000x reference)
import functools

import jax
import jax.numpy as jnp
from jax import lax
from jax.experimental import pallas as pl
from jax.experimental.pallas import tpu as pltpu
from jax.experimental.pallas import tpu_sc as plsc

DIM = 1024
SEQ = 4096
NUM_CORES = 2
NUM_SUBCORES = 16
NW = NUM_CORES * NUM_SUBCORES
ROWS_W = SEQ // NW
CHUNK = 32
NCHUNK = ROWS_W // CHUNK


@functools.partial(
    pl.kernel,
    mesh=plsc.VectorSubcoreMesh(core_axis_name="c", subcore_axis_name="s"),
    out_type=jax.ShapeDtypeStruct((SEQ, DIM), jnp.float32),
    scratch_types=[
        pltpu.VMEM((ROWS_W,), jnp.int32),
        pltpu.VMEM((CHUNK, DIM), jnp.float32),
        pltpu.VMEM((CHUNK, DIM), jnp.float32),
        pltpu.SemaphoreType.DMA,
        pltpu.SemaphoreType.DMA,
        pltpu.SemaphoreType.DMA,
        pltpu.SemaphoreType.DMA,
    ],
)
def _sc_slice(idx_hbm, table_hbm, out_hbm, idx_v, buf0, buf1,
              sg0, sg1, sp0, sp1):
    wid = lax.axis_index("s") * NUM_CORES + lax.axis_index("c")
    base = wid * ROWS_W
    pltpu.sync_copy(idx_hbm.at[pl.ds(base, ROWS_W)], idx_v)

    bufs = (buf0, buf1)
    gsems = (sg0, sg1)
    psems = (sp0, sp1)

    def gather(c):
        return pltpu.async_copy(
            table_hbm.at[idx_v.at[pl.ds(c * CHUNK, CHUNK)]],
            bufs[c % 2], gsems[c % 2])

    def put(c):
        return pltpu.async_copy(
            bufs[c % 2], out_hbm.at[pl.ds(base + c * CHUNK, CHUNK)],
            psems[c % 2])

    gs = [gather(0), gather(1)]
    ps = [None, None]
    for c in range(NCHUNK):
        gs[c % 2].wait()
        ps[c % 2] = put(c)
        if c + 2 < NCHUNK:
            ps[c % 2].wait()
            gs[c % 2] = gather(c + 2)
    ps[0].wait()
    ps[1].wait()


def kernel(seq_len, emb_weight):
    start = jnp.asarray(seq_len, jnp.int32) - SEQ
    idx = start + lax.iota(jnp.int32, SEQ)
    return _sc_slice(idx, emb_weight)
